# baseline (speedup 1.0000x reference)
import functools

import jax
import jax.numpy as jnp
from jax import lax
from jax.experimental import pallas as pl
from jax.experimental.pallas import tpu as pltpu
from jax.experimental.pallas import tpu_sc as plsc

_NC = 2
_NS = 16
_NW = _NC * _NS

_L = 16
_CHUNK = 128


def _rot():
    iota = lax.iota(jnp.int32, _L)
    return [lax.bitwise_and(iota + r, _L - 1) for r in range(_L)]


def _make_transpose(E, V):
    groups = V // _CHUNK
    per_w = groups // _NW
    extra = groups - per_w * _NW
    tail = V - groups * _CHUNK

    mesh = plsc.VectorSubcoreMesh(core_axis_name="c", subcore_axis_name="s")

    @functools.partial(
        pl.kernel,
        mesh=mesh,
        compiler_params=pltpu.CompilerParams(needs_layout_passes=False),
        out_type=jax.ShapeDtypeStruct((V // 2, 2 * E), jnp.float32),
        scratch_types=[
            pltpu.VMEM((2, E, _CHUNK), jnp.float32),
            pltpu.VMEM((2, _CHUNK // 2, 2 * E), jnp.float32),
            pltpu.SemaphoreType.DMA,
            pltpu.SemaphoreType.DMA,
            pltpu.SemaphoreType.DMA,
            pltpu.SemaphoreType.DMA,
        ],
    )
    def transpose_kernel(tab_hbm, tail_hbm, out_hbm, buf_v, outb_v,
                         si0, si1, so0, so1):
        wid = lax.axis_index("s") * _NC + lax.axis_index("c")
        base = wid * per_w
        si = (si0, si1)
        so = (so0, so1)
        iota = lax.iota(jnp.int32, _L)
        colrot = _rot()

        def stage_in(g, b):
            pltpu.async_copy(
                tab_hbm.at[:, pl.ds(g * _CHUNK, _CHUNK)], buf_v.at[b], si[b])

        def wait_in(b):
            pltpu.make_async_copy(
                tab_hbm.at[:, pl.ds(0, _CHUNK)], buf_v.at[b], si[b]).wait()

        def wait_out(b):
            pltpu.make_async_copy(
                outb_v.at[b], out_hbm.at[pl.ds(0, _CHUNK // 2)], so[b]).wait()

        def transpose_group(b):
            @plsc.parallel_loop(0, _CHUNK // _L, unroll=2)
            def t_block(tq):
                t0 = tq * _L
                tv = t0 + iota
                srows = lax.shift_right_logical(tv, 1)
                tpar = lax.shift_left(lax.bitwise_and(tv, 1), 6)
                for m in range(E // _L):
                    evs = [m * _L + colrot[r] for r in range(_L)]
                    vals = [plsc.load_gather(buf_v.at[b], [ev, tv])
                            for ev in evs]
                    for ev, val in zip(evs, vals):
                        plsc.store_scatter(outb_v.at[b],
                                           [srows, tpar + ev], val)

        stage_in(base, 0)

        def pair(k, carry):
            for b in range(2):
                g = base + 2 * k + b

                @pl.when(2 * k + b + 1 < per_w)
                def _():
                    stage_in(g + 1, 1 - b)
                wait_in(b)

                @pl.when(2 * k + b >= 2)
                def _():
                    wait_out(b)
                transpose_group(b)
                pltpu.async_copy(
                    outb_v.at[b],
                    out_hbm.at[pl.ds(g * (_CHUNK // 2), _CHUNK // 2)],
                    so[b],
                )
            return carry

        lax.fori_loop(0, per_w // 2, pair, 0)
        for b in range(2):
            @pl.when(per_w >= 2 - b)
            def _():
                wait_out(b)

        @pl.when(wid < extra)
        def _():
            g = _NW * per_w + wid
            stage_in(g, 0)
            wait_in(0)
            transpose_group(0)
            pltpu.async_copy(
                outb_v.at[0],
                out_hbm.at[pl.ds(g * (_CHUNK // 2), _CHUNK // 2)], so[0])
            wait_out(0)

        if tail:
            @pl.when(wid == _NW - 1)
            def _():
                pltpu.sync_copy(tail_hbm, outb_v.at[0, pl.ds(0, tail // 2)])
                pltpu.sync_copy(
                    outb_v.at[0, pl.ds(0, tail // 2)],
                    out_hbm.at[pl.ds(groups * (_CHUNK // 2), tail // 2)],
                )

    return transpose_kernel


def _make_gather(S, B, E):
    per_w = S * B // _NW
    assert B // _NW == _CHUNK

    mesh = plsc.VectorSubcoreMesh(core_axis_name="c", subcore_axis_name="s")

    @functools.partial(
        pl.kernel,
        mesh=mesh,
        compiler_params=pltpu.CompilerParams(needs_layout_passes=False),
        out_type=jax.ShapeDtypeStruct((S, E, B), jnp.float32),
        scratch_types=[
            pltpu.VMEM((per_w,), jnp.int32),
            pltpu.VMEM((per_w,), jnp.int32),
            pltpu.VMEM((per_w,), jnp.int32),
            pltpu.VMEM((2, _CHUNK, 2 * E), jnp.float32),
            pltpu.VMEM((2, E, _CHUNK), jnp.float32),
            pltpu.SemaphoreType.DMA,
            pltpu.SemaphoreType.DMA,
            pltpu.SemaphoreType.DMA,
            pltpu.SemaphoreType.DMA,
        ],
    )
    def gather_kernel(table_hbm, idx_hbm, out_hbm, idx_v, pair_v, off_v,
                      buf_v, outb_v, si0, si1, so0, so1):
        wid = lax.axis_index("s") * _NC + lax.axis_index("c")
        pltpu.sync_copy(idx_hbm.at[pl.ds(wid * per_w, per_w)], idx_v)
        si = (si0, si1)
        so = (so0, so1)
        iota = lax.iota(jnp.int32, _L)
        colrot = _rot()

        @plsc.parallel_loop(0, per_w // _L, unroll=4)
        def prep(k):
            v = idx_v[pl.ds(k * _L, _L)]
            pair_v[pl.ds(k * _L, _L)] = lax.shift_right_logical(v, 1)
            off_v[pl.ds(k * _L, _L)] = lax.shift_left(
                lax.bitwise_and(v, 1), 6)

        def stage_in(s, b):
            pltpu.async_copy(
                table_hbm.at[pair_v.at[pl.ds(s * _CHUNK, _CHUNK)]],
                buf_v.at[b], si[b])

        def wait_in(b):
            pltpu.make_async_copy(
                table_hbm.at[pl.ds(0, _CHUNK)], buf_v.at[b], si[b]).wait()

        def wait_out(b):
            pltpu.make_async_copy(
                outb_v.at[b], out_hbm.at[0, :, pl.ds(0, _CHUNK)], so[b]).wait()

        def extract(s, b):
            @plsc.parallel_loop(0, _CHUNK // _L, unroll=2)
            def t_block(tq):
                t0 = tq * _L
                tv = t0 + iota
                offs = off_v[pl.ds(s * _CHUNK + t0, _L)]
                for m in range(E // _L):
                    evs = [m * _L + colrot[r] for r in range(_L)]
                    vals = [plsc.load_gather(buf_v.at[b], [tv, offs + ev])
                            for ev in evs]
                    for ev, val in zip(evs, vals):
                        plsc.store_scatter(outb_v.at[b], [ev, tv], val)

        stage_in(0, 0)

        def pair(k, carry):
            for b in range(2):
                s = 2 * k + b

                @pl.when(s + 1 < S)
                def _():
                    stage_in(s + 1, 1 - b)
                wait_in(b)

                @pl.when(s >= 2)
                def _():
                    wait_out(b)
                extract(s, b)
                pltpu.async_copy(
                    outb_v.at[b],
                    out_hbm.at[s, :, pl.ds(wid * _CHUNK, _CHUNK)], so[b])
            return carry

        lax.fori_loop(0, S // 2, pair, 0)
        for b in range(2):
            wait_out(b)

    return gather_kernel


def kernel(input, table):
    seq, batch = input.shape
    vocab, embed = table.shape
    ntail = vocab % _CHUNK
    tail2 = table[vocab - ntail:, :].reshape(ntail // 2, 2 * embed)
    table2 = _make_transpose(embed, vocab)(table.T, tail2)
    idxp = (input.reshape(seq, _NW, _CHUNK)
            .transpose(1, 0, 2)
            .reshape(-1))
    out = _make_gather(seq, batch, embed)(table2, idxp)
    return out.transpose(0, 2, 1)


# speedup vs baseline: 1.7692x; 1.7692x over previous
import functools

import jax
import jax.numpy as jnp
from jax import lax
from jax.experimental import pallas as pl
from jax.experimental.pallas import tpu as pltpu
from jax.experimental.pallas import tpu_sc as plsc

_NC = 2
_NS = 16
_NW = _NC * _NS

_L = 16
_CHUNK = 128


def _rot():
    iota = lax.iota(jnp.int32, _L)
    return [lax.bitwise_and(iota + r, _L - 1) for r in range(_L)]


def _make_transpose(E, V):
    groups = V // _CHUNK
    per_w = groups // _NW
    extra = groups - per_w * _NW
    tail = V - groups * _CHUNK

    mesh = plsc.VectorSubcoreMesh(core_axis_name="c", subcore_axis_name="s")

    @functools.partial(
        pl.kernel,
        mesh=mesh,
        compiler_params=pltpu.CompilerParams(needs_layout_passes=False),
        out_type=jax.ShapeDtypeStruct((V // 2, 2 * E), jnp.float32),
        scratch_types=[
            pltpu.VMEM((2, E, _CHUNK), jnp.float32),
            pltpu.VMEM((2, _CHUNK // 2, 2 * E), jnp.float32),
            pltpu.SemaphoreType.DMA,
            pltpu.SemaphoreType.DMA,
            pltpu.SemaphoreType.DMA,
            pltpu.SemaphoreType.DMA,
        ],
    )
    def transpose_kernel(tab_hbm, tail_hbm, out_hbm, buf_v, outb_v,
                         si0, si1, so0, so1):
        wid = lax.axis_index("s") * _NC + lax.axis_index("c")
        base = wid * per_w
        si = (si0, si1)
        so = (so0, so1)
        iota = lax.iota(jnp.int32, _L)
        colrot = _rot()

        def stage_in(g, b):
            pltpu.async_copy(
                tab_hbm.at[:, pl.ds(g * _CHUNK, _CHUNK)], buf_v.at[b], si[b])

        def wait_in(b):
            pltpu.make_async_copy(
                tab_hbm.at[:, pl.ds(0, _CHUNK)], buf_v.at[b], si[b]).wait()

        def wait_out(b):
            pltpu.make_async_copy(
                outb_v.at[b], out_hbm.at[pl.ds(0, _CHUNK // 2)], so[b]).wait()

        def transpose_group(b):
            @plsc.parallel_loop(0, _CHUNK // _L)
            def t_block(tq):
                t0 = tq * _L
                tv = t0 + iota
                srows = lax.shift_right_logical(tv, 1)
                tpar = lax.shift_left(lax.bitwise_and(tv, 1), 6)
                for m in range(E // _L):
                    evs = [m * _L + colrot[r] for r in range(_L)]
                    vals = [plsc.load_gather(buf_v.at[b], [ev, tv])
                            for ev in evs]
                    for ev, val in zip(evs, vals):
                        plsc.store_scatter(outb_v.at[b],
                                           [srows, tpar + ev], val)

        stage_in(base, 0)

        def pair(k, carry):
            for b in range(2):
                g = base + 2 * k + b

                @pl.when(2 * k + b + 1 < per_w)
                def _():
                    stage_in(g + 1, 1 - b)
                wait_in(b)

                @pl.when(2 * k + b >= 2)
                def _():
                    wait_out(b)
                transpose_group(b)
                pltpu.async_copy(
                    outb_v.at[b],
                    out_hbm.at[pl.ds(g * (_CHUNK // 2), _CHUNK // 2)],
                    so[b],
                )
            return carry

        lax.fori_loop(0, per_w // 2, pair, 0)
        for b in range(2):
            @pl.when(per_w >= 2 - b)
            def _():
                wait_out(b)

        @pl.when(wid < extra)
        def _():
            g = _NW * per_w + wid
            stage_in(g, 0)
            wait_in(0)
            transpose_group(0)
            pltpu.async_copy(
                outb_v.at[0],
                out_hbm.at[pl.ds(g * (_CHUNK // 2), _CHUNK // 2)], so[0])
            wait_out(0)

        if tail:
            @pl.when(wid == _NW - 1)
            def _():
                pltpu.sync_copy(tail_hbm, outb_v.at[0, pl.ds(0, tail // 2)])
                pltpu.sync_copy(
                    outb_v.at[0, pl.ds(0, tail // 2)],
                    out_hbm.at[pl.ds(groups * (_CHUNK // 2), tail // 2)],
                )

    return transpose_kernel


def _make_gather(S, B, E):
    per_w = S * B // _NW
    assert B // _NW == _CHUNK

    mesh = plsc.VectorSubcoreMesh(core_axis_name="c", subcore_axis_name="s")

    @functools.partial(
        pl.kernel,
        mesh=mesh,
        compiler_params=pltpu.CompilerParams(needs_layout_passes=False),
        out_type=jax.ShapeDtypeStruct((S, E, B), jnp.float32),
        scratch_types=[
            pltpu.VMEM((per_w,), jnp.int32),
            pltpu.VMEM((per_w,), jnp.int32),
            pltpu.VMEM((per_w,), jnp.int32),
            pltpu.VMEM((2, _CHUNK, 2 * E), jnp.float32),
            pltpu.VMEM((2, E, _CHUNK), jnp.float32),
            pltpu.SemaphoreType.DMA,
            pltpu.SemaphoreType.DMA,
            pltpu.SemaphoreType.DMA,
            pltpu.SemaphoreType.DMA,
        ],
    )
    def gather_kernel(table_hbm, idx_hbm, out_hbm, idx_v, pair_v, off_v,
                      buf_v, outb_v, si0, si1, so0, so1):
        wid = lax.axis_index("s") * _NC + lax.axis_index("c")
        pltpu.sync_copy(idx_hbm.at[pl.ds(wid * per_w, per_w)], idx_v)
        si = (si0, si1)
        so = (so0, so1)
        iota = lax.iota(jnp.int32, _L)
        colrot = _rot()

        @plsc.parallel_loop(0, per_w // _L, unroll=4)
        def prep(k):
            v = idx_v[pl.ds(k * _L, _L)]
            pair_v[pl.ds(k * _L, _L)] = lax.shift_right_logical(v, 1)
            off_v[pl.ds(k * _L, _L)] = lax.shift_left(
                lax.bitwise_and(v, 1), 6)

        def stage_in(s, b):
            pltpu.async_copy(
                table_hbm.at[pair_v.at[pl.ds(s * _CHUNK, _CHUNK)]],
                buf_v.at[b], si[b])

        def wait_in(b):
            pltpu.make_async_copy(
                table_hbm.at[pl.ds(0, _CHUNK)], buf_v.at[b], si[b]).wait()

        def wait_out(b):
            pltpu.make_async_copy(
                outb_v.at[b], out_hbm.at[0, :, pl.ds(0, _CHUNK)], so[b]).wait()

        def extract(s, b):
            @plsc.parallel_loop(0, _CHUNK // _L)
            def t_block(tq):
                t0 = tq * _L
                tv = t0 + iota
                offs = off_v[pl.ds(s * _CHUNK + t0, _L)]
                for m in range(E // _L):
                    evs = [m * _L + colrot[r] for r in range(_L)]
                    vals = [plsc.load_gather(buf_v.at[b], [tv, offs + ev])
                            for ev in evs]
                    for ev, val in zip(evs, vals):
                        plsc.store_scatter(outb_v.at[b], [ev, tv], val)

        stage_in(0, 0)

        def pair(k, carry):
            for b in range(2):
                s = 2 * k + b

                @pl.when(s + 1 < S)
                def _():
                    stage_in(s + 1, 1 - b)
                wait_in(b)

                @pl.when(s >= 2)
                def _():
                    wait_out(b)
                extract(s, b)
                pltpu.async_copy(
                    outb_v.at[b],
                    out_hbm.at[s, :, pl.ds(wid * _CHUNK, _CHUNK)], so[b])
            return carry

        lax.fori_loop(0, S // 2, pair, 0)
        for b in range(2):
            wait_out(b)

    return gather_kernel


def kernel(input, table):
    seq, batch = input.shape
    vocab, embed = table.shape
    ntail = vocab % _CHUNK
    tail2 = table[vocab - ntail:, :].reshape(ntail // 2, 2 * embed)
    table2 = _make_transpose(embed, vocab)(table.T, tail2)
    idxp = (input.reshape(seq, _NW, _CHUNK)
            .transpose(1, 0, 2)
            .reshape(-1))
    out = _make_gather(seq, batch, embed)(table2, idxp)
    return out.transpose(0, 2, 1)
